# Initial kernel scaffold; baseline (speedup 1.0000x reference)
import functools

import jax
import jax.numpy as jnp
from jax import lax
from jax.experimental import pallas as pl
from jax.experimental.pallas import tpu as pltpu
from jax.experimental.pallas import tpu_sc as plsc

NN = 50000
F = 64
LANE = 128
NS = 16
CHUNK_R = 8
CK = CHUNK_R * LANE

EE = 800000
EP = -(-EE // (NS * CK)) * (NS * CK)
RT = EP // LANE
RPS = RT // NS
NCHUNK = RPS // CHUNK_R

HALF = NN // 2
ACC_ROWS = HALF + 8
ZROWS = ACC_ROWS // NS
HIST = NN + 48
HZ = HIST // NS

_mesh = plsc.VectorSubcoreMesh(core_axis_name="c", subcore_axis_name="s")


@functools.partial(
    pl.kernel,
    out_type=(jax.ShapeDtypeStruct((2, HIST), jnp.float32),
              jax.ShapeDtypeStruct((2, RT, LANE), jnp.int32)),
    mesh=_mesh,
    scratch_types=[
        pltpu.VMEM_SHARED((HIST,), jnp.float32),
        pltpu.VMEM((CHUNK_R, LANE), jnp.int32),
        pltpu.VMEM((CHUNK_R, LANE), jnp.int32),
        pltpu.VMEM((CHUNK_R, LANE), jnp.int32),
        pltpu.VMEM((LANE,), jnp.float32),
    ],
)
def _sc_pre(srch, dsth, zh, deg2, dloc, hist_sh, ichunk, dchunk, lchunk, ones_v):
    c = lax.axis_index("c")
    s = lax.axis_index("s")
    base = c * HALF
    for i in range(LANE // 16):
        ones_v[pl.ds(i * 16, 16)] = jnp.full((16,), 1.0, jnp.float32)
    pltpu.sync_copy(zh.at[pl.ds(s * HZ, HZ)], hist_sh.at[pl.ds(s * HZ, HZ)])
    plsc.subcore_barrier()

    def chunk(k, carry):
        r0 = s * RPS + k * CHUNK_R
        pltpu.sync_copy(dsth.at[pl.ds(r0, CHUNK_R)], dchunk)

        @pl.when(c == 0)
        def _():
            pltpu.sync_copy(srch.at[pl.ds(r0, CHUNK_R)], ichunk)

        @pl.when(c != 0)
        def _():
            pltpu.sync_copy(dsth.at[pl.ds(r0, CHUNK_R)], ichunk)

        for r in range(CHUNK_R):
            pltpu.sync_copy(ones_v, hist_sh.at[ichunk.at[r]], add=True)
        for r in range(CHUNK_R):
            for j in range(LANE // 16):
                d = dchunk[r, pl.ds(j * 16, 16)]
                dl = d - base
                inr = (dl >= 0) & (dl < HALF)
                li = jnp.where(inr, dl, HALF + (d & 7))
                lchunk[r, pl.ds(j * 16, 16)] = li
        pltpu.sync_copy(lchunk, dloc.at[c, pl.ds(r0, CHUNK_R)])
        return carry

    lax.fori_loop(0, NCHUNK, chunk, 0)
    plsc.subcore_barrier()

    @pl.when(s == 0)
    def _():
        pltpu.sync_copy(hist_sh, deg2.at[c])


@functools.partial(
    pl.kernel,
    out_type=jax.ShapeDtypeStruct((2, ACC_ROWS, F), jnp.float32),
    mesh=_mesh,
    scratch_types=[
        pltpu.VMEM_SHARED((ACC_ROWS, F), jnp.float32),
        pltpu.VMEM((CHUNK_R, LANE), jnp.int32),
        pltpu.VMEM((CHUNK_R, LANE), jnp.int32),
        pltpu.VMEM((CK, F), jnp.float32),
        pltpu.SemaphoreType.DMA,
    ],
)
def _sc_agg(hs, srcg, dloc, za, out, acc_sh, idx_v, loc_v, rows_v, sem):
    c = lax.axis_index("c")
    s = lax.axis_index("s")
    pltpu.sync_copy(za.at[pl.ds(s * ZROWS, ZROWS)],
                    acc_sh.at[pl.ds(s * ZROWS, ZROWS)])
    plsc.subcore_barrier()

    def chunk(k, carry):
        r0 = s * RPS + k * CHUNK_R
        pltpu.sync_copy(srcg.at[pl.ds(r0, CHUNK_R)], idx_v)
        pltpu.sync_copy(dloc.at[c, pl.ds(r0, CHUNK_R)], loc_v)
        descs = [
            pltpu.async_copy(hs.at[idx_v.at[j]],
                             rows_v.at[pl.ds(j * LANE, LANE)], sem)
            for j in range(CHUNK_R)
        ]
        for d_ in descs:
            d_.wait()
        for j in range(CHUNK_R):
            pltpu.sync_copy(rows_v.at[pl.ds(j * LANE, LANE)],
                            acc_sh.at[loc_v.at[j]], add=True)
        return carry

    lax.fori_loop(0, NCHUNK, chunk, 0)
    plsc.subcore_barrier()
    pltpu.sync_copy(acc_sh.at[pl.ds(s * ZROWS, ZROWS)],
                    out.at[c, pl.ds(s * ZROWS, ZROWS)])


BLK = 1000
GRID = NN // BLK
BPH = HALF // BLK


def _tc_in_body(x_ref, w_ref, b_ref, degs_ref, o_ref):
    ns = lax.rsqrt(jnp.maximum(degs_ref[...], 1.0))
    h = jnp.tanh(jnp.dot(x_ref[...], w_ref[...],
                         preferred_element_type=jnp.float32) + b_ref[...])
    o_ref[...] = h * ns


def _tc_mid_body(agg_ref, degd_ref, w_ref, b_ref, degs_ref, o_ref):
    nd = lax.rsqrt(jnp.maximum(degd_ref[...], 1.0))
    a = agg_ref[0] * nd
    h = jnp.tanh(jnp.dot(a, w_ref[...],
                         preferred_element_type=jnp.float32) + b_ref[...])
    o_ref[...] = h * lax.rsqrt(jnp.maximum(degs_ref[...], 1.0))


def _tc_fin_body(agg_ref, degd_ref, w2_ref, b2_ref, wd_ref, bd_ref, wc_ref,
                 bc_ref, o_ref):
    nd = lax.rsqrt(jnp.maximum(degd_ref[...], 1.0))
    a = agg_ref[0] * nd
    h = jnp.dot(a, w2_ref[...], preferred_element_type=jnp.float32) + b2_ref[...]
    t = jnp.tanh(jnp.dot(h, wd_ref[...],
                         preferred_element_type=jnp.float32) + bd_ref[...])
    o_ref[...] = jnp.dot(t, wc_ref[...],
                         preferred_element_type=jnp.float32) + bc_ref[...]


def _row_spec(cols):
    return pl.BlockSpec((BLK, cols), lambda i: (i, 0))


def _full_spec(r, cols):
    return pl.BlockSpec((r, cols), lambda i: (0, 0))


_agg_spec = pl.BlockSpec((1, BLK, F), lambda i: (i // BPH, i % BPH, 0))


def _tc_in(x, w, b, degs):
    return pl.pallas_call(
        _tc_in_body,
        grid=(GRID,),
        in_specs=[_row_spec(x.shape[1]), _full_spec(*w.shape),
                  _full_spec(1, F), _row_spec(1)],
        out_specs=_row_spec(F),
        out_shape=jax.ShapeDtypeStruct((NN, F), jnp.float32),
    )(x, w, b, degs)


def _tc_mid(agg3, degd, w, b, degs):
    return pl.pallas_call(
        _tc_mid_body,
        grid=(GRID,),
        in_specs=[_agg_spec, _row_spec(1), _full_spec(F, F),
                  _full_spec(1, F), _row_spec(1)],
        out_specs=_row_spec(F),
        out_shape=jax.ShapeDtypeStruct((NN, F), jnp.float32),
    )(agg3, degd, w, b, degs)


def _tc_fin(agg3, degd, w2, b2, wd, bd, wc, bc):
    return pl.pallas_call(
        _tc_fin_body,
        grid=(GRID,),
        in_specs=[_agg_spec, _row_spec(1), _full_spec(F, F), _full_spec(1, F),
                  _full_spec(F, F), _full_spec(1, F), _full_spec(F, 45),
                  _full_spec(1, 45)],
        out_specs=_row_spec(45),
        out_shape=jax.ShapeDtypeStruct((NN, 45), jnp.float32),
    )(agg3, degd, w2, b2, wd, bd, wc, bc)


def kernel(x, edge_index, W_in, b_in, W0, b0, W1, b1, W2, b2, Wd, bd, Wc, bc):
    src = edge_index[0].astype(jnp.int32)
    dst = edge_index[1].astype(jnp.int32)
    pad = EP - EE
    srcg = jnp.concatenate([src, jnp.zeros((pad,), jnp.int32)]).reshape(RT, LANE)
    srch = jnp.concatenate([src, jnp.full((pad,), NN, jnp.int32)]).reshape(RT, LANE)
    dsth = jnp.concatenate([dst, jnp.full((pad,), NN, jnp.int32)]).reshape(RT, LANE)
    zh = jnp.zeros((HIST,), jnp.float32)
    za = jnp.zeros((ACC_ROWS, F), jnp.float32)

    deg2, dloc = _sc_pre(srch, dsth, zh)
    out_deg = deg2[0, :NN].reshape(NN, 1)
    in_deg = deg2[1, :NN].reshape(NN, 1)

    hs = _tc_in(x, W_in, b_in.reshape(1, F), out_deg)
    for (w, b) in ((W0, b0), (W1, b1)):
        agg3 = _sc_agg(hs, srcg, dloc, za)
        hs = _tc_mid(agg3, in_deg, w, b.reshape(1, F), out_deg)
    agg3 = _sc_agg(hs, srcg, dloc, za)
    return _tc_fin(agg3, in_deg, W2, b2.reshape(1, F), Wd, bd.reshape(1, F),
                   Wc, bc.reshape(1, 45))


# speedup vs baseline: 4.8998x; 4.8998x over previous
import functools

import jax
import jax.numpy as jnp
from jax import lax
from jax.experimental import pallas as pl
from jax.experimental.pallas import tpu as pltpu
from jax.experimental.pallas import tpu_sc as plsc

NN = 50000
F = 64
LANE = 128
NS = 16
CHUNK_R = 8
CK = CHUNK_R * LANE

EE = 800000
EP = -(-EE // (NS * CK)) * (NS * CK)
RT = EP // LANE
RPS = RT // NS
NCHUNK = RPS // CHUNK_R

HALF = NN // 2
ACC_ROWS = HALF + 88
ZROWS = ACC_ROWS // NS

ACR = 2
ACK = ACR * LANE
ANCHUNK = RPS // ACR
HIST = 51200
HZ = HIST // NS

_mesh = plsc.VectorSubcoreMesh(core_axis_name="c", subcore_axis_name="s")


@functools.partial(
    pl.kernel,
    out_type=(jax.ShapeDtypeStruct((2, HIST), jnp.float32),
              jax.ShapeDtypeStruct((2, RT, LANE), jnp.int32)),
    mesh=_mesh,
    compiler_params=pltpu.CompilerParams(use_tc_tiling_on_sc=False),
    scratch_types=[
        pltpu.VMEM_SHARED((HIST,), jnp.float32),
        pltpu.VMEM((CHUNK_R, LANE), jnp.int32),
        pltpu.VMEM((CHUNK_R, LANE), jnp.int32),
        pltpu.VMEM((CHUNK_R, LANE), jnp.int32),
        pltpu.VMEM((LANE,), jnp.float32),
        pltpu.VMEM((HZ,), jnp.float32),
    ],
)
def _sc_pre(srch, dsth, zh, deg2, dloc, hist_sh, ichunk, dchunk, lchunk, ones_v,
            stage_v):
    c = lax.axis_index("c")
    s = lax.axis_index("s")
    base = c * HALF
    for i in range(LANE // 16):
        ones_v[pl.ds(i * 16, 16)] = jnp.full((16,), 1.0, jnp.float32)
    pltpu.sync_copy(zh.at[pl.ds(s * HZ, HZ)], stage_v)
    pltpu.sync_copy(stage_v, hist_sh.at[pl.ds(s * HZ, HZ)])
    plsc.subcore_barrier()

    def chunk(k, carry):
        r0 = s * RPS + k * CHUNK_R
        pltpu.sync_copy(dsth.at[pl.ds(r0, CHUNK_R)], dchunk)

        @pl.when(c == 0)
        def _():
            pltpu.sync_copy(srch.at[pl.ds(r0, CHUNK_R)], ichunk)

        @pl.when(c != 0)
        def _():
            pltpu.sync_copy(dsth.at[pl.ds(r0, CHUNK_R)], ichunk)

        for r in range(CHUNK_R):
            pltpu.sync_copy(ones_v, hist_sh.at[ichunk.at[r]], add=True)
        for r in range(CHUNK_R):
            for j in range(LANE // 16):
                d = dchunk[r, pl.ds(j * 16, 16)]
                dl = d - base
                inr = (dl >= 0) & (dl < HALF)
                li = jnp.where(inr, dl, HALF + (d & 7))
                lchunk[r, pl.ds(j * 16, 16)] = li
        pltpu.sync_copy(lchunk, dloc.at[c, pl.ds(r0, CHUNK_R)])
        return carry

    lax.fori_loop(0, NCHUNK, chunk, 0)
    plsc.subcore_barrier()
    pltpu.sync_copy(hist_sh.at[pl.ds(s * HZ, HZ)], stage_v)
    pltpu.sync_copy(stage_v, deg2.at[c, pl.ds(s * HZ, HZ)])


@functools.partial(
    pl.kernel,
    out_type=jax.ShapeDtypeStruct((2, ACC_ROWS, F), jnp.float32),
    mesh=_mesh,
    compiler_params=pltpu.CompilerParams(use_tc_tiling_on_sc=False),
    scratch_types=[
        pltpu.VMEM_SHARED((ACC_ROWS, F), jnp.float32),
        pltpu.VMEM((ACR, LANE), jnp.int32),
        pltpu.VMEM((ACR, LANE), jnp.int32),
        pltpu.VMEM((ACK, F), jnp.float32),
        pltpu.SemaphoreType.DMA,
    ],
)
def _sc_agg(hs, srcg, dloc, za, out, acc_sh, idx_v, loc_v, rows_v, sem):
    c = lax.axis_index("c")
    s = lax.axis_index("s")
    pltpu.sync_copy(za, rows_v)

    def zero(k, carry):
        pltpu.sync_copy(rows_v, acc_sh.at[pl.ds(s * ZROWS + k * ACK, ACK)])
        return carry

    lax.fori_loop(0, ZROWS // ACK, zero, 0)
    pltpu.sync_copy(rows_v.at[pl.ds(0, ZROWS % ACK)],
                    acc_sh.at[pl.ds(s * ZROWS + (ZROWS // ACK) * ACK,
                                    ZROWS % ACK)])
    plsc.subcore_barrier()

    def chunk(k, carry):
        r0 = s * RPS + k * ACR
        pltpu.sync_copy(srcg.at[pl.ds(r0, ACR)], idx_v)
        pltpu.sync_copy(dloc.at[c, pl.ds(r0, ACR)], loc_v)
        descs = [
            pltpu.async_copy(hs.at[idx_v.at[j]],
                             rows_v.at[pl.ds(j * LANE, LANE)], sem)
            for j in range(ACR)
        ]
        for d_ in descs:
            d_.wait()
        for j in range(ACR):
            pltpu.sync_copy(rows_v.at[pl.ds(j * LANE, LANE)],
                            acc_sh.at[loc_v.at[j]], add=True)
        return carry

    lax.fori_loop(0, ANCHUNK, chunk, 0)
    plsc.subcore_barrier()

    def copyout(k, carry):
        pltpu.sync_copy(acc_sh.at[pl.ds(s * ZROWS + k * ACK, ACK)], rows_v)
        pltpu.sync_copy(rows_v, out.at[c, pl.ds(s * ZROWS + k * ACK, ACK)])
        return carry

    lax.fori_loop(0, ZROWS // ACK, copyout, 0)
    pltpu.sync_copy(acc_sh.at[pl.ds(s * ZROWS + (ZROWS // ACK) * ACK,
                                    ZROWS % ACK)], rows_v.at[pl.ds(0, ZROWS % ACK)])
    pltpu.sync_copy(rows_v.at[pl.ds(0, ZROWS % ACK)],
                    out.at[c, pl.ds(s * ZROWS + (ZROWS // ACK) * ACK,
                                    ZROWS % ACK)])


BLK = 1000
GRID = NN // BLK
BPH = HALF // BLK


def _tc_in_body(x_ref, w_ref, b_ref, degs_ref, o_ref):
    ns = lax.rsqrt(jnp.maximum(degs_ref[...], 1.0))
    h = jnp.tanh(jnp.dot(x_ref[...], w_ref[...],
                         preferred_element_type=jnp.float32) + b_ref[...])
    o_ref[...] = h * ns


def _tc_mid_body(agg_ref, degd_ref, w_ref, b_ref, degs_ref, o_ref):
    nd = lax.rsqrt(jnp.maximum(degd_ref[...], 1.0))
    a = agg_ref[0] * nd
    h = jnp.tanh(jnp.dot(a, w_ref[...],
                         preferred_element_type=jnp.float32) + b_ref[...])
    o_ref[...] = h * lax.rsqrt(jnp.maximum(degs_ref[...], 1.0))


def _tc_fin_body(agg_ref, degd_ref, w2_ref, b2_ref, wd_ref, bd_ref, wc_ref,
                 bc_ref, o_ref):
    nd = lax.rsqrt(jnp.maximum(degd_ref[...], 1.0))
    a = agg_ref[0] * nd
    h = jnp.dot(a, w2_ref[...], preferred_element_type=jnp.float32) + b2_ref[...]
    t = jnp.tanh(jnp.dot(h, wd_ref[...],
                         preferred_element_type=jnp.float32) + bd_ref[...])
    o_ref[...] = jnp.dot(t, wc_ref[...],
                         preferred_element_type=jnp.float32) + bc_ref[...]


def _row_spec(cols):
    return pl.BlockSpec((BLK, cols), lambda i: (i, 0))


def _full_spec(r, cols):
    return pl.BlockSpec((r, cols), lambda i: (0, 0))


_agg_spec = pl.BlockSpec((1, BLK, F), lambda i: (i // BPH, i % BPH, 0))


def _tc_in(x, w, b, degs):
    return pl.pallas_call(
        _tc_in_body,
        grid=(GRID,),
        in_specs=[_row_spec(x.shape[1]), _full_spec(*w.shape),
                  _full_spec(1, F), _row_spec(1)],
        out_specs=_row_spec(F),
        out_shape=jax.ShapeDtypeStruct((NN, F), jnp.float32),
    )(x, w, b, degs)


def _tc_mid(agg3, degd, w, b, degs):
    return pl.pallas_call(
        _tc_mid_body,
        grid=(GRID,),
        in_specs=[_agg_spec, _row_spec(1), _full_spec(F, F),
                  _full_spec(1, F), _row_spec(1)],
        out_specs=_row_spec(F),
        out_shape=jax.ShapeDtypeStruct((NN, F), jnp.float32),
    )(agg3, degd, w, b, degs)


def _tc_fin(agg3, degd, w2, b2, wd, bd, wc, bc):
    return pl.pallas_call(
        _tc_fin_body,
        grid=(GRID,),
        in_specs=[_agg_spec, _row_spec(1), _full_spec(F, F), _full_spec(1, F),
                  _full_spec(F, F), _full_spec(1, F), _full_spec(F, 45),
                  _full_spec(1, 45)],
        out_specs=_row_spec(45),
        out_shape=jax.ShapeDtypeStruct((NN, 45), jnp.float32),
    )(agg3, degd, w2, b2, wd, bd, wc, bc)


def kernel(x, edge_index, W_in, b_in, W0, b0, W1, b1, W2, b2, Wd, bd, Wc, bc):
    src = edge_index[0].astype(jnp.int32)
    dst = edge_index[1].astype(jnp.int32)
    pad = EP - EE
    srcg = jnp.concatenate([src, jnp.zeros((pad,), jnp.int32)]).reshape(RT, LANE)
    srch = jnp.concatenate([src, jnp.full((pad,), NN, jnp.int32)]).reshape(RT, LANE)
    dsth = jnp.concatenate([dst, jnp.full((pad,), NN, jnp.int32)]).reshape(RT, LANE)
    zh = jnp.zeros((HIST,), jnp.float32)
    za = jnp.zeros((ACK, F), jnp.float32)

    deg2, dloc = _sc_pre(srch, dsth, zh)
    out_deg = deg2[0, :NN].reshape(NN, 1)
    in_deg = deg2[1, :NN].reshape(NN, 1)

    hs = _tc_in(x, W_in, b_in.reshape(1, F), out_deg)
    for (w, b) in ((W0, b0), (W1, b1)):
        agg3 = _sc_agg(hs, srcg, dloc, za)
        hs = _tc_mid(agg3, in_deg, w, b.reshape(1, F), out_deg)
    agg3 = _sc_agg(hs, srcg, dloc, za)
    return _tc_fin(agg3, in_deg, W2, b2.reshape(1, F), Wd, bd.reshape(1, F),
                   Wc, bc.reshape(1, 45))


# speedup vs baseline: 5.3053x; 1.0827x over previous
import functools

import jax
import jax.numpy as jnp
from jax import lax
from jax.experimental import pallas as pl
from jax.experimental.pallas import tpu as pltpu
from jax.experimental.pallas import tpu_sc as plsc

NN = 50000
F = 64
LANE = 128
NS = 16
CHUNK_R = 8
CK = CHUNK_R * LANE

EE = 800000
EP = -(-EE // (NS * CK)) * (NS * CK)
RT = EP // LANE
RPS = RT // NS
NCHUNK = RPS // CHUNK_R

HALF = NN // 2
ACC_ROWS = HALF + 88
ZROWS = ACC_ROWS // NS

NB = 2
ACK = NB * LANE
HIST = 51200
HZ = HIST // NS

_mesh = plsc.VectorSubcoreMesh(core_axis_name="c", subcore_axis_name="s")


@functools.partial(
    pl.kernel,
    out_type=(jax.ShapeDtypeStruct((2, HIST), jnp.float32),
              jax.ShapeDtypeStruct((2, RT, LANE), jnp.int32)),
    mesh=_mesh,
    compiler_params=pltpu.CompilerParams(use_tc_tiling_on_sc=False),
    scratch_types=[
        pltpu.VMEM_SHARED((HIST,), jnp.float32),
        pltpu.VMEM((CHUNK_R, LANE), jnp.int32),
        pltpu.VMEM((CHUNK_R, LANE), jnp.int32),
        pltpu.VMEM((CHUNK_R, LANE), jnp.int32),
        pltpu.VMEM((LANE,), jnp.float32),
        pltpu.VMEM((HZ,), jnp.float32),
    ],
)
def _sc_pre(srch, dsth, zh, deg2, dloc, hist_sh, ichunk, dchunk, lchunk, ones_v,
            stage_v):
    c = lax.axis_index("c")
    s = lax.axis_index("s")
    base = c * HALF
    for i in range(LANE // 16):
        ones_v[pl.ds(i * 16, 16)] = jnp.full((16,), 1.0, jnp.float32)
    pltpu.sync_copy(zh.at[pl.ds(s * HZ, HZ)], stage_v)
    pltpu.sync_copy(stage_v, hist_sh.at[pl.ds(s * HZ, HZ)])
    plsc.subcore_barrier()

    def chunk(k, carry):
        r0 = s * RPS + k * CHUNK_R
        pltpu.sync_copy(dsth.at[pl.ds(r0, CHUNK_R)], dchunk)

        @pl.when(c == 0)
        def _():
            pltpu.sync_copy(srch.at[pl.ds(r0, CHUNK_R)], ichunk)

        @pl.when(c != 0)
        def _():
            pltpu.sync_copy(dsth.at[pl.ds(r0, CHUNK_R)], ichunk)

        for r in range(CHUNK_R):
            pltpu.sync_copy(ones_v, hist_sh.at[ichunk.at[r]], add=True)
        for r in range(CHUNK_R):
            for j in range(LANE // 16):
                d = dchunk[r, pl.ds(j * 16, 16)]
                dl = d - base
                inr = (dl >= 0) & (dl < HALF)
                li = jnp.where(inr, dl, HALF + (d & 7))
                lchunk[r, pl.ds(j * 16, 16)] = li
        pltpu.sync_copy(lchunk, dloc.at[c, pl.ds(r0, CHUNK_R)])
        return carry

    lax.fori_loop(0, NCHUNK, chunk, 0)
    plsc.subcore_barrier()
    pltpu.sync_copy(hist_sh.at[pl.ds(s * HZ, HZ)], stage_v)
    pltpu.sync_copy(stage_v, deg2.at[c, pl.ds(s * HZ, HZ)])


@functools.partial(
    pl.kernel,
    out_type=jax.ShapeDtypeStruct((2, ACC_ROWS, F), jnp.float32),
    mesh=_mesh,
    compiler_params=pltpu.CompilerParams(use_tc_tiling_on_sc=False),
    scratch_types=[
        pltpu.VMEM_SHARED((ACC_ROWS, F), jnp.float32),
        pltpu.VMEM((NB, LANE), jnp.int32),
        pltpu.VMEM((NB, LANE), jnp.int32),
        pltpu.VMEM((ACK, F), jnp.float32),
        pltpu.SemaphoreType.DMA,
        pltpu.SemaphoreType.DMA,
    ],
)
def _sc_agg(hs, srcg, dloc, za, out, acc_sh, idx_v, loc_v, rows_v, sem0, sem1):
    c = lax.axis_index("c")
    s = lax.axis_index("s")
    pltpu.sync_copy(za, rows_v)

    def zero(k, carry):
        pltpu.sync_copy(rows_v, acc_sh.at[pl.ds(s * ZROWS + k * ACK, ACK)])
        return carry

    lax.fori_loop(0, ZROWS // ACK, zero, 0)
    pltpu.sync_copy(rows_v.at[pl.ds(0, ZROWS % ACK)],
                    acc_sh.at[pl.ds(s * ZROWS + (ZROWS // ACK) * ACK,
                                    ZROWS % ACK)])
    plsc.subcore_barrier()

    sems = (sem0, sem1)

    def _fetch(g, b):
        r0 = s * RPS + g
        pltpu.sync_copy(srcg.at[pl.ds(r0, 1)], idx_v.at[pl.ds(b, 1)])
        pltpu.sync_copy(dloc.at[c, pl.ds(r0, 1)], loc_v.at[pl.ds(b, 1)])
        pltpu.async_copy(hs.at[idx_v.at[b]],
                         rows_v.at[pl.ds(b * LANE, LANE)], sems[b])

    for b in range(NB):
        _fetch(b, b)

    def chunk(t, carry):
        for b in range(NB):
            g = NB * t + b
            pltpu.make_async_copy(hs.at[idx_v.at[b]],
                                  rows_v.at[pl.ds(b * LANE, LANE)],
                                  sems[b]).wait()
            pltpu.sync_copy(rows_v.at[pl.ds(b * LANE, LANE)],
                            acc_sh.at[loc_v.at[b]], add=True)

            @pl.when(g + NB < RPS)
            def _():
                _fetch(g + NB, b)
        return carry

    lax.fori_loop(0, RPS // NB, chunk, 0)
    plsc.subcore_barrier()

    def copyout(k, carry):
        pltpu.sync_copy(acc_sh.at[pl.ds(s * ZROWS + k * ACK, ACK)], rows_v)
        pltpu.sync_copy(rows_v, out.at[c, pl.ds(s * ZROWS + k * ACK, ACK)])
        return carry

    lax.fori_loop(0, ZROWS // ACK, copyout, 0)
    pltpu.sync_copy(acc_sh.at[pl.ds(s * ZROWS + (ZROWS // ACK) * ACK,
                                    ZROWS % ACK)], rows_v.at[pl.ds(0, ZROWS % ACK)])
    pltpu.sync_copy(rows_v.at[pl.ds(0, ZROWS % ACK)],
                    out.at[c, pl.ds(s * ZROWS + (ZROWS // ACK) * ACK,
                                    ZROWS % ACK)])


BLK = 1000
GRID = NN // BLK
BPH = HALF // BLK


def _tc_in_body(x_ref, w_ref, b_ref, degs_ref, o_ref):
    ns = lax.rsqrt(jnp.maximum(degs_ref[...], 1.0))
    h = jnp.tanh(jnp.dot(x_ref[...], w_ref[...],
                         preferred_element_type=jnp.float32) + b_ref[...])
    o_ref[...] = h * ns


def _tc_mid_body(agg_ref, degd_ref, w_ref, b_ref, degs_ref, o_ref):
    nd = lax.rsqrt(jnp.maximum(degd_ref[...], 1.0))
    a = agg_ref[0] * nd
    h = jnp.tanh(jnp.dot(a, w_ref[...],
                         preferred_element_type=jnp.float32) + b_ref[...])
    o_ref[...] = h * lax.rsqrt(jnp.maximum(degs_ref[...], 1.0))


def _tc_fin_body(agg_ref, degd_ref, w2_ref, b2_ref, wd_ref, bd_ref, wc_ref,
                 bc_ref, o_ref):
    nd = lax.rsqrt(jnp.maximum(degd_ref[...], 1.0))
    a = agg_ref[0] * nd
    h = jnp.dot(a, w2_ref[...], preferred_element_type=jnp.float32) + b2_ref[...]
    t = jnp.tanh(jnp.dot(h, wd_ref[...],
                         preferred_element_type=jnp.float32) + bd_ref[...])
    o_ref[...] = jnp.dot(t, wc_ref[...],
                         preferred_element_type=jnp.float32) + bc_ref[...]


def _row_spec(cols):
    return pl.BlockSpec((BLK, cols), lambda i: (i, 0))


def _full_spec(r, cols):
    return pl.BlockSpec((r, cols), lambda i: (0, 0))


_agg_spec = pl.BlockSpec((1, BLK, F), lambda i: (i // BPH, i % BPH, 0))


def _tc_in(x, w, b, degs):
    return pl.pallas_call(
        _tc_in_body,
        grid=(GRID,),
        in_specs=[_row_spec(x.shape[1]), _full_spec(*w.shape),
                  _full_spec(1, F), _row_spec(1)],
        out_specs=_row_spec(F),
        out_shape=jax.ShapeDtypeStruct((NN, F), jnp.float32),
    )(x, w, b, degs)


def _tc_mid(agg3, degd, w, b, degs):
    return pl.pallas_call(
        _tc_mid_body,
        grid=(GRID,),
        in_specs=[_agg_spec, _row_spec(1), _full_spec(F, F),
                  _full_spec(1, F), _row_spec(1)],
        out_specs=_row_spec(F),
        out_shape=jax.ShapeDtypeStruct((NN, F), jnp.float32),
    )(agg3, degd, w, b, degs)


def _tc_fin(agg3, degd, w2, b2, wd, bd, wc, bc):
    return pl.pallas_call(
        _tc_fin_body,
        grid=(GRID,),
        in_specs=[_agg_spec, _row_spec(1), _full_spec(F, F), _full_spec(1, F),
                  _full_spec(F, F), _full_spec(1, F), _full_spec(F, 45),
                  _full_spec(1, 45)],
        out_specs=_row_spec(45),
        out_shape=jax.ShapeDtypeStruct((NN, 45), jnp.float32),
    )(agg3, degd, w2, b2, wd, bd, wc, bc)


def kernel(x, edge_index, W_in, b_in, W0, b0, W1, b1, W2, b2, Wd, bd, Wc, bc):
    src = edge_index[0].astype(jnp.int32)
    dst = edge_index[1].astype(jnp.int32)
    pad = EP - EE
    srcg = jnp.concatenate([src, jnp.zeros((pad,), jnp.int32)]).reshape(RT, LANE)
    srch = jnp.concatenate([src, jnp.full((pad,), NN, jnp.int32)]).reshape(RT, LANE)
    dsth = jnp.concatenate([dst, jnp.full((pad,), NN, jnp.int32)]).reshape(RT, LANE)
    zh = jnp.zeros((HIST,), jnp.float32)
    za = jnp.zeros((ACK, F), jnp.float32)

    deg2, dloc = _sc_pre(srch, dsth, zh)
    out_deg = deg2[0, :NN].reshape(NN, 1)
    in_deg = deg2[1, :NN].reshape(NN, 1)

    hs = _tc_in(x, W_in, b_in.reshape(1, F), out_deg)
    for (w, b) in ((W0, b0), (W1, b1)):
        agg3 = _sc_agg(hs, srcg, dloc, za)
        hs = _tc_mid(agg3, in_deg, w, b.reshape(1, F), out_deg)
    agg3 = _sc_agg(hs, srcg, dloc, za)
    return _tc_fin(agg3, in_deg, W2, b2.reshape(1, F), Wd, bd.reshape(1, F),
                   Wc, bc.reshape(1, 45))


# speedup vs baseline: 8.1981x; 1.5453x over previous
import functools

import jax
import jax.numpy as jnp
from jax import lax
from jax.experimental import pallas as pl
from jax.experimental.pallas import tpu as pltpu
from jax.experimental.pallas import tpu_sc as plsc

NN = 50000
F = 64
LANE = 128
NS = 16
CHUNK_R = 8
CK = CHUNK_R * LANE

EE = 800000
EP = -(-EE // (NS * CK)) * (NS * CK)
RT = EP // LANE
RPS = RT // NS
NCHUNK = RPS // CHUNK_R

HALF = NN // 2
ACC_ROWS = HALF + 88
ZROWS = ACC_ROWS // NS

NB = 2
ACK = NB * LANE
HIST = 51200
HZ = HIST // NS

_mesh = plsc.VectorSubcoreMesh(core_axis_name="c", subcore_axis_name="s")


SROWS = 16


@functools.partial(
    pl.kernel,
    out_type=(jax.ShapeDtypeStruct((2, HIST), jnp.float32),
              jax.ShapeDtypeStruct((2, RT, LANE), jnp.int32),
              jax.ShapeDtypeStruct((2, RT, LANE), jnp.int32),
              jax.ShapeDtypeStruct((2, NS, 16), jnp.int32)),
    mesh=_mesh,
    compiler_params=pltpu.CompilerParams(use_tc_tiling_on_sc=False,
                                         needs_layout_passes=False),
    scratch_types=[
        pltpu.VMEM_SHARED((HIST,), jnp.float32),
        pltpu.VMEM((CHUNK_R, LANE), jnp.int32),
        pltpu.VMEM((CHUNK_R, LANE), jnp.int32),
        pltpu.VMEM((CHUNK_R, LANE), jnp.int32),
        pltpu.VMEM((SROWS, LANE), jnp.int32),
        pltpu.VMEM((SROWS, LANE), jnp.int32),
        pltpu.VMEM((LANE,), jnp.float32),
        pltpu.VMEM((HZ,), jnp.float32),
        pltpu.VMEM((16,), jnp.int32),
    ],
)
def _sc_pre(srch, dsth, srcg, zh, deg2, cg, cl, cnt, hist_sh, ichunk, dchunk,
            schunk, sg, sl, ones_v, stage_v, cbuf):
    c = lax.axis_index("c")
    s = lax.axis_index("s")
    base = c * HALF
    for i in range(LANE // 16):
        ones_v[pl.ds(i * 16, 16)] = jnp.full((16,), 1.0, jnp.float32)
    pltpu.sync_copy(zh.at[pl.ds(s * HZ, HZ)], stage_v)
    pltpu.sync_copy(stage_v, hist_sh.at[pl.ds(s * HZ, HZ)])
    plsc.subcore_barrier()

    def chunk(k, carry):
        w, fl = carry
        r0 = s * RPS + k * CHUNK_R
        pltpu.sync_copy(dsth.at[pl.ds(r0, CHUNK_R)], dchunk)
        pltpu.sync_copy(srcg.at[pl.ds(r0, CHUNK_R)], schunk)

        @pl.when(c == 0)
        def _():
            pltpu.sync_copy(srch.at[pl.ds(r0, CHUNK_R)], ichunk)

        @pl.when(c != 0)
        def _():
            pltpu.sync_copy(dsth.at[pl.ds(r0, CHUNK_R)], ichunk)

        for r in range(CHUNK_R):
            pltpu.sync_copy(ones_v, hist_sh.at[ichunk.at[r]], add=True)
        for r in range(CHUNK_R):
            for j in range(LANE // 16):
                d = dchunk[r, pl.ds(j * 16, 16)]
                sv = schunk[r, pl.ds(j * 16, 16)]
                dl = d - base
                m = (dl >= 0) & (dl < HALF)
                cs = plsc.cumsum(m.astype(jnp.int32))
                pos = lax.broadcast_in_dim(w, (16,), ()) + cs - 1
                prow = lax.shift_right_logical(pos, 7) & (SROWS - 1)
                pcol = pos & (LANE - 1)
                plsc.store_scatter(sg, [prow, pcol], sv, mask=m)
                plsc.store_scatter(sl, [prow, pcol], dl, mask=m)
                w = w + jnp.sum(m.astype(jnp.int32))

        def _flush(args):
            w_, fl_ = args
            fr = fl_ & (SROWS - 1)
            pltpu.sync_copy(sg.at[pl.ds(fr, CHUNK_R)],
                            cg.at[c, pl.ds(s * RPS + fl_, CHUNK_R)])
            pltpu.sync_copy(sl.at[pl.ds(fr, CHUNK_R)],
                            cl.at[c, pl.ds(s * RPS + fl_, CHUNK_R)])
            return w_, fl_ + CHUNK_R

        return lax.cond(w - fl * LANE >= CHUNK_R * LANE, _flush,
                        lambda a: a, (w, fl))

    w, fl = lax.fori_loop(0, NCHUNK, chunk, (jnp.int32(0), jnp.int32(0)))

    for j in range(CHUNK_R):
        idx = (lax.broadcast_in_dim(w, (16,), ()) + (j * 16)
               + lax.iota(jnp.int32, 16))
        prow = lax.shift_right_logical(idx, 7) & (SROWS - 1)
        pcol = idx & (LANE - 1)
        plsc.store_scatter(sg, [prow, pcol], jnp.zeros((16,), jnp.int32))
        plsc.store_scatter(sl, [prow, pcol], jnp.full((16,), HALF, jnp.int32))
    nrows = lax.shift_right_logical(w + LANE - 1, 7)

    def frow(r, carry):
        fr = r & (SROWS - 1)
        pltpu.sync_copy(sg.at[pl.ds(fr, 1)], cg.at[c, pl.ds(s * RPS + r, 1)])
        pltpu.sync_copy(sl.at[pl.ds(fr, 1)], cl.at[c, pl.ds(s * RPS + r, 1)])
        return carry

    lax.fori_loop(fl, nrows, frow, 0)
    cbuf[...] = lax.broadcast_in_dim(nrows, (16,), ())
    pltpu.sync_copy(cbuf, cnt.at[c, s])
    plsc.subcore_barrier()
    pltpu.sync_copy(hist_sh.at[pl.ds(s * HZ, HZ)], stage_v)
    pltpu.sync_copy(stage_v, deg2.at[c, pl.ds(s * HZ, HZ)])


@functools.partial(
    pl.kernel,
    out_type=jax.ShapeDtypeStruct((2, ACC_ROWS, F), jnp.float32),
    mesh=_mesh,
    compiler_params=pltpu.CompilerParams(use_tc_tiling_on_sc=False,
                                         needs_layout_passes=False),
    scratch_types=[
        pltpu.VMEM_SHARED((ACC_ROWS, F), jnp.float32),
        pltpu.VMEM((NB, LANE), jnp.int32),
        pltpu.VMEM((NB, LANE), jnp.int32),
        pltpu.VMEM((ACK, F), jnp.float32),
        pltpu.VMEM((16,), jnp.int32),
        pltpu.SemaphoreType.DMA,
        pltpu.SemaphoreType.DMA,
    ],
)
def _sc_agg(hs, cg, cl, cnt, za, out, acc_sh, idx_v, loc_v, rows_v, cbuf,
            sem0, sem1):
    c = lax.axis_index("c")
    s = lax.axis_index("s")
    pltpu.sync_copy(za, rows_v)

    def zero(k, carry):
        pltpu.sync_copy(rows_v, acc_sh.at[pl.ds(s * ZROWS + k * ACK, ACK)])
        return carry

    lax.fori_loop(0, ZROWS // ACK, zero, 0)
    pltpu.sync_copy(rows_v.at[pl.ds(0, ZROWS % ACK)],
                    acc_sh.at[pl.ds(s * ZROWS + (ZROWS // ACK) * ACK,
                                    ZROWS % ACK)])
    plsc.subcore_barrier()

    sems = (sem0, sem1)
    pltpu.sync_copy(cnt.at[c, s], cbuf)
    nr = jnp.max(cbuf[...])

    def _fetch(g, b):
        r0 = s * RPS + g
        pltpu.sync_copy(cg.at[c, pl.ds(r0, 1)], idx_v.at[pl.ds(b, 1)])
        pltpu.sync_copy(cl.at[c, pl.ds(r0, 1)], loc_v.at[pl.ds(b, 1)])
        pltpu.async_copy(hs.at[idx_v.at[b]],
                         rows_v.at[pl.ds(b * LANE, LANE)], sems[b])

    for b in range(NB):
        @pl.when(b < nr)
        def _(b=b):
            _fetch(b, b)

    def chunk(t, carry):
        for b in range(NB):
            g = NB * t + b

            @pl.when(g < nr)
            def _(g=g, b=b):
                pltpu.make_async_copy(hs.at[idx_v.at[b]],
                                      rows_v.at[pl.ds(b * LANE, LANE)],
                                      sems[b]).wait()
                pltpu.sync_copy(rows_v.at[pl.ds(b * LANE, LANE)],
                                acc_sh.at[loc_v.at[b]], add=True)

                @pl.when(g + NB < nr)
                def _():
                    _fetch(g + NB, b)
        return carry

    lax.fori_loop(0, lax.div(nr + NB - 1, NB), chunk, 0)
    plsc.subcore_barrier()

    def copyout(k, carry):
        pltpu.sync_copy(acc_sh.at[pl.ds(s * ZROWS + k * ACK, ACK)], rows_v)
        pltpu.sync_copy(rows_v, out.at[c, pl.ds(s * ZROWS + k * ACK, ACK)])
        return carry

    lax.fori_loop(0, ZROWS // ACK, copyout, 0)
    pltpu.sync_copy(acc_sh.at[pl.ds(s * ZROWS + (ZROWS // ACK) * ACK,
                                    ZROWS % ACK)], rows_v.at[pl.ds(0, ZROWS % ACK)])
    pltpu.sync_copy(rows_v.at[pl.ds(0, ZROWS % ACK)],
                    out.at[c, pl.ds(s * ZROWS + (ZROWS // ACK) * ACK,
                                    ZROWS % ACK)])


BLK = 1000
GRID = NN // BLK
BPH = HALF // BLK


def _tc_in_body(x_ref, w_ref, b_ref, degs_ref, o_ref):
    ns = lax.rsqrt(jnp.maximum(degs_ref[...], 1.0))
    h = jnp.tanh(jnp.dot(x_ref[...], w_ref[...],
                         preferred_element_type=jnp.float32) + b_ref[...])
    o_ref[...] = h * ns


def _tc_mid_body(agg_ref, degd_ref, w_ref, b_ref, degs_ref, o_ref):
    nd = lax.rsqrt(jnp.maximum(degd_ref[...], 1.0))
    a = agg_ref[0] * nd
    h = jnp.tanh(jnp.dot(a, w_ref[...],
                         preferred_element_type=jnp.float32) + b_ref[...])
    o_ref[...] = h * lax.rsqrt(jnp.maximum(degs_ref[...], 1.0))


def _tc_fin_body(agg_ref, degd_ref, w2_ref, b2_ref, wd_ref, bd_ref, wc_ref,
                 bc_ref, o_ref):
    nd = lax.rsqrt(jnp.maximum(degd_ref[...], 1.0))
    a = agg_ref[0] * nd
    h = jnp.dot(a, w2_ref[...], preferred_element_type=jnp.float32) + b2_ref[...]
    t = jnp.tanh(jnp.dot(h, wd_ref[...],
                         preferred_element_type=jnp.float32) + bd_ref[...])
    o_ref[...] = jnp.dot(t, wc_ref[...],
                         preferred_element_type=jnp.float32) + bc_ref[...]


def _row_spec(cols):
    return pl.BlockSpec((BLK, cols), lambda i: (i, 0))


def _full_spec(r, cols):
    return pl.BlockSpec((r, cols), lambda i: (0, 0))


_agg_spec = pl.BlockSpec((1, BLK, F), lambda i: (i // BPH, i % BPH, 0))


def _tc_in(x, w, b, degs):
    return pl.pallas_call(
        _tc_in_body,
        grid=(GRID,),
        in_specs=[_row_spec(x.shape[1]), _full_spec(*w.shape),
                  _full_spec(1, F), _row_spec(1)],
        out_specs=_row_spec(F),
        out_shape=jax.ShapeDtypeStruct((NN, F), jnp.float32),
    )(x, w, b, degs)


def _tc_mid(agg3, degd, w, b, degs):
    return pl.pallas_call(
        _tc_mid_body,
        grid=(GRID,),
        in_specs=[_agg_spec, _row_spec(1), _full_spec(F, F),
                  _full_spec(1, F), _row_spec(1)],
        out_specs=_row_spec(F),
        out_shape=jax.ShapeDtypeStruct((NN, F), jnp.float32),
    )(agg3, degd, w, b, degs)


def _tc_fin(agg3, degd, w2, b2, wd, bd, wc, bc):
    return pl.pallas_call(
        _tc_fin_body,
        grid=(GRID,),
        in_specs=[_agg_spec, _row_spec(1), _full_spec(F, F), _full_spec(1, F),
                  _full_spec(F, F), _full_spec(1, F), _full_spec(F, 45),
                  _full_spec(1, 45)],
        out_specs=_row_spec(45),
        out_shape=jax.ShapeDtypeStruct((NN, 45), jnp.float32),
    )(agg3, degd, w2, b2, wd, bd, wc, bc)


def kernel(x, edge_index, W_in, b_in, W0, b0, W1, b1, W2, b2, Wd, bd, Wc, bc):
    src = edge_index[0].astype(jnp.int32)
    dst = edge_index[1].astype(jnp.int32)
    pad = EP - EE
    srcg = jnp.concatenate([src, jnp.zeros((pad,), jnp.int32)]).reshape(RT, LANE)
    srch = jnp.concatenate([src, jnp.full((pad,), NN, jnp.int32)]).reshape(RT, LANE)
    dsth = jnp.concatenate([dst, jnp.full((pad,), NN, jnp.int32)]).reshape(RT, LANE)
    zh = jnp.zeros((HIST,), jnp.float32)
    za = jnp.zeros((ACK, F), jnp.float32)

    deg2, cg, cl, cnt = _sc_pre(srch, dsth, srcg, zh)
    out_deg = deg2[0, :NN].reshape(NN, 1)
    in_deg = deg2[1, :NN].reshape(NN, 1)

    hs = _tc_in(x, W_in, b_in.reshape(1, F), out_deg)
    for (w, b) in ((W0, b0), (W1, b1)):
        agg3 = _sc_agg(hs, cg, cl, cnt, za)
        hs = _tc_mid(agg3, in_deg, w, b.reshape(1, F), out_deg)
    agg3 = _sc_agg(hs, cg, cl, cnt, za)
    return _tc_fin(agg3, in_deg, W2, b2.reshape(1, F), Wd, bd.reshape(1, F),
                   Wc, bc.reshape(1, 45))


# speedup vs baseline: 10.1672x; 1.2402x over previous
import functools

import jax
import jax.numpy as jnp
from jax import lax
from jax.experimental import pallas as pl
from jax.experimental.pallas import tpu as pltpu
from jax.experimental.pallas import tpu_sc as plsc

NN = 50000
F = 64
LANE = 128
NS = 16
CHUNK_R = 8
CK = CHUNK_R * LANE

EE = 800000
EP = -(-EE // (NS * CK)) * (NS * CK)
RT = EP // LANE
RPS = RT // NS
NCHUNK = RPS // CHUNK_R

HALF = NN // 2
ACC_ROWS = HALF + 88
ZROWS = ACC_ROWS // NS

NB = 2
ACK = NB * LANE
IRING = 32
HIST = 51200
HZ = HIST // NS

_mesh = plsc.VectorSubcoreMesh(core_axis_name="c", subcore_axis_name="s")


SROWS = 16


@functools.partial(
    pl.kernel,
    out_type=(jax.ShapeDtypeStruct((2, HIST), jnp.float32),
              jax.ShapeDtypeStruct((2, RT, LANE), jnp.int32),
              jax.ShapeDtypeStruct((2, RT, LANE), jnp.int32),
              jax.ShapeDtypeStruct((2, NS, 16), jnp.int32)),
    mesh=_mesh,
    compiler_params=pltpu.CompilerParams(use_tc_tiling_on_sc=False,
                                         needs_layout_passes=False),
    scratch_types=[
        pltpu.VMEM_SHARED((HIST,), jnp.float32),
        pltpu.VMEM((CHUNK_R, LANE), jnp.int32),
        pltpu.VMEM((CHUNK_R, LANE), jnp.int32),
        pltpu.VMEM((CHUNK_R, LANE), jnp.int32),
        pltpu.VMEM((SROWS, LANE), jnp.int32),
        pltpu.VMEM((SROWS, LANE), jnp.int32),
        pltpu.VMEM((LANE,), jnp.float32),
        pltpu.VMEM((HZ,), jnp.float32),
        pltpu.VMEM((16,), jnp.int32),
    ],
)
def _sc_pre(srch, dsth, srcg, zh, deg2, cg, cl, cnt, hist_sh, ichunk, dchunk,
            schunk, sg, sl, ones_v, stage_v, cbuf):
    c = lax.axis_index("c")
    s = lax.axis_index("s")
    base = c * HALF
    for i in range(LANE // 16):
        ones_v[pl.ds(i * 16, 16)] = jnp.full((16,), 1.0, jnp.float32)
    pltpu.sync_copy(zh.at[pl.ds(s * HZ, HZ)], stage_v)
    pltpu.sync_copy(stage_v, hist_sh.at[pl.ds(s * HZ, HZ)])
    plsc.subcore_barrier()

    def chunk(k, carry):
        w, fl = carry
        r0 = s * RPS + k * CHUNK_R
        pltpu.sync_copy(dsth.at[pl.ds(r0, CHUNK_R)], dchunk)
        pltpu.sync_copy(srcg.at[pl.ds(r0, CHUNK_R)], schunk)

        @pl.when(c == 0)
        def _():
            pltpu.sync_copy(srch.at[pl.ds(r0, CHUNK_R)], ichunk)

        @pl.when(c != 0)
        def _():
            pltpu.sync_copy(dsth.at[pl.ds(r0, CHUNK_R)], ichunk)

        for r in range(CHUNK_R):
            pltpu.sync_copy(ones_v, hist_sh.at[ichunk.at[r]], add=True)
        for r in range(CHUNK_R):
            for j in range(LANE // 16):
                d = dchunk[r, pl.ds(j * 16, 16)]
                sv = schunk[r, pl.ds(j * 16, 16)]
                dl = d - base
                m = (dl >= 0) & (dl < HALF)
                cs = plsc.cumsum(m.astype(jnp.int32))
                pos = lax.broadcast_in_dim(w, (16,), ()) + cs - 1
                prow = lax.shift_right_logical(pos, 7) & (SROWS - 1)
                pcol = pos & (LANE - 1)
                plsc.store_scatter(sg, [prow, pcol], sv, mask=m)
                plsc.store_scatter(sl, [prow, pcol], dl, mask=m)
                w = w + jnp.sum(m.astype(jnp.int32))

        def _flush(args):
            w_, fl_ = args
            fr = fl_ & (SROWS - 1)
            pltpu.sync_copy(sg.at[pl.ds(fr, CHUNK_R)],
                            cg.at[c, pl.ds(s * RPS + fl_, CHUNK_R)])
            pltpu.sync_copy(sl.at[pl.ds(fr, CHUNK_R)],
                            cl.at[c, pl.ds(s * RPS + fl_, CHUNK_R)])
            return w_, fl_ + CHUNK_R

        return lax.cond(w - fl * LANE >= CHUNK_R * LANE, _flush,
                        lambda a: a, (w, fl))

    w, fl = lax.fori_loop(0, NCHUNK, chunk, (jnp.int32(0), jnp.int32(0)))

    for j in range(CHUNK_R):
        idx = (lax.broadcast_in_dim(w, (16,), ()) + (j * 16)
               + lax.iota(jnp.int32, 16))
        prow = lax.shift_right_logical(idx, 7) & (SROWS - 1)
        pcol = idx & (LANE - 1)
        plsc.store_scatter(sg, [prow, pcol], jnp.zeros((16,), jnp.int32))
        plsc.store_scatter(sl, [prow, pcol], jnp.full((16,), HALF, jnp.int32))
    nrows = lax.shift_right_logical(w + LANE - 1, 7)

    def frow(r, carry):
        fr = r & (SROWS - 1)
        pltpu.sync_copy(sg.at[pl.ds(fr, 1)], cg.at[c, pl.ds(s * RPS + r, 1)])
        pltpu.sync_copy(sl.at[pl.ds(fr, 1)], cl.at[c, pl.ds(s * RPS + r, 1)])
        return carry

    lax.fori_loop(fl, nrows, frow, 0)
    cbuf[...] = lax.broadcast_in_dim(nrows, (16,), ())
    pltpu.sync_copy(cbuf, cnt.at[c, s])
    plsc.subcore_barrier()
    pltpu.sync_copy(hist_sh.at[pl.ds(s * HZ, HZ)], stage_v)
    pltpu.sync_copy(stage_v, deg2.at[c, pl.ds(s * HZ, HZ)])


@functools.partial(
    pl.kernel,
    out_type=jax.ShapeDtypeStruct((2, ACC_ROWS, F), jnp.float32),
    mesh=_mesh,
    compiler_params=pltpu.CompilerParams(use_tc_tiling_on_sc=False,
                                         needs_layout_passes=False),
    scratch_types=[
        pltpu.VMEM_SHARED((ACC_ROWS, F), jnp.float32),
        pltpu.VMEM((IRING, LANE), jnp.int32),
        pltpu.VMEM((IRING, LANE), jnp.int32),
        pltpu.VMEM((ACK, F), jnp.float32),
        pltpu.VMEM((16,), jnp.int32),
        pltpu.SemaphoreType.DMA,
        pltpu.SemaphoreType.DMA,
    ],
)
def _sc_agg(hs, cg, cl, cnt, za, out, acc_sh, idx_v, loc_v, rows_v, cbuf,
            sem0, sem1):
    c = lax.axis_index("c")
    s = lax.axis_index("s")
    pltpu.sync_copy(za, rows_v)

    def zero(k, carry):
        pltpu.sync_copy(rows_v, acc_sh.at[pl.ds(s * ZROWS + k * ACK, ACK)])
        return carry

    lax.fori_loop(0, ZROWS // ACK, zero, 0)
    pltpu.sync_copy(rows_v.at[pl.ds(0, ZROWS % ACK)],
                    acc_sh.at[pl.ds(s * ZROWS + (ZROWS // ACK) * ACK,
                                    ZROWS % ACK)])
    plsc.subcore_barrier()

    sems = (sem0, sem1)
    pltpu.sync_copy(cnt.at[c, s], cbuf)
    nr = jnp.max(cbuf[...])
    tbase = s * RPS

    def _ifill(g0):
        r0 = jnp.minimum(tbase + g0, RT - 16)
        sl_ = g0 & (IRING - 1)
        pltpu.sync_copy(cg.at[c, pl.ds(r0, 16)], idx_v.at[pl.ds(sl_, 16)])
        pltpu.sync_copy(cl.at[c, pl.ds(r0, 16)], loc_v.at[pl.ds(sl_, 16)])

    _ifill(jnp.int32(0))
    _ifill(jnp.int32(16))

    def _issue(g, b):
        pltpu.async_copy(hs.at[idx_v.at[g & (IRING - 1)]],
                         rows_v.at[pl.ds(b * LANE, LANE)], sems[b])

    for b in range(NB):
        @pl.when(b < nr)
        def _(b=b):
            _issue(jnp.int32(b), b)

    def chunk(t, carry):
        for b in range(NB):
            g = NB * t + b

            @pl.when(g < nr)
            def _(g=g, b=b):
                pltpu.make_async_copy(hs.at[idx_v.at[g & (IRING - 1)]],
                                      rows_v.at[pl.ds(b * LANE, LANE)],
                                      sems[b]).wait()
                pltpu.sync_copy(rows_v.at[pl.ds(b * LANE, LANE)],
                                acc_sh.at[loc_v.at[g & (IRING - 1)]], add=True)
                if b == 0:
                    @pl.when((g & 15) == 0)
                    def _():
                        _ifill(g + 16)

                @pl.when(g + NB < nr)
                def _():
                    _issue(g + NB, b)
        return carry

    lax.fori_loop(0, lax.div(nr + NB - 1, NB), chunk, 0)
    plsc.subcore_barrier()

    def copyout(k, carry):
        pltpu.sync_copy(acc_sh.at[pl.ds(s * ZROWS + k * ACK, ACK)], rows_v)
        pltpu.sync_copy(rows_v, out.at[c, pl.ds(s * ZROWS + k * ACK, ACK)])
        return carry

    lax.fori_loop(0, ZROWS // ACK, copyout, 0)
    pltpu.sync_copy(acc_sh.at[pl.ds(s * ZROWS + (ZROWS // ACK) * ACK,
                                    ZROWS % ACK)], rows_v.at[pl.ds(0, ZROWS % ACK)])
    pltpu.sync_copy(rows_v.at[pl.ds(0, ZROWS % ACK)],
                    out.at[c, pl.ds(s * ZROWS + (ZROWS // ACK) * ACK,
                                    ZROWS % ACK)])


BLK = 1000
GRID = NN // BLK
BPH = HALF // BLK


def _tc_in_body(x_ref, w_ref, b_ref, degs_ref, o_ref):
    ns = lax.rsqrt(jnp.maximum(degs_ref[...], 1.0))
    h = jnp.tanh(jnp.dot(x_ref[...], w_ref[...],
                         preferred_element_type=jnp.float32) + b_ref[...])
    o_ref[...] = h * ns


def _tc_mid_body(agg_ref, degd_ref, w_ref, b_ref, degs_ref, o_ref):
    nd = lax.rsqrt(jnp.maximum(degd_ref[...], 1.0))
    a = agg_ref[0] * nd
    h = jnp.tanh(jnp.dot(a, w_ref[...],
                         preferred_element_type=jnp.float32) + b_ref[...])
    o_ref[...] = h * lax.rsqrt(jnp.maximum(degs_ref[...], 1.0))


def _tc_fin_body(agg_ref, degd_ref, w2_ref, b2_ref, wd_ref, bd_ref, wc_ref,
                 bc_ref, o_ref):
    nd = lax.rsqrt(jnp.maximum(degd_ref[...], 1.0))
    a = agg_ref[0] * nd
    h = jnp.dot(a, w2_ref[...], preferred_element_type=jnp.float32) + b2_ref[...]
    t = jnp.tanh(jnp.dot(h, wd_ref[...],
                         preferred_element_type=jnp.float32) + bd_ref[...])
    o_ref[...] = jnp.dot(t, wc_ref[...],
                         preferred_element_type=jnp.float32) + bc_ref[...]


def _row_spec(cols):
    return pl.BlockSpec((BLK, cols), lambda i: (i, 0))


def _full_spec(r, cols):
    return pl.BlockSpec((r, cols), lambda i: (0, 0))


_agg_spec = pl.BlockSpec((1, BLK, F), lambda i: (i // BPH, i % BPH, 0))


def _tc_in(x, w, b, degs):
    return pl.pallas_call(
        _tc_in_body,
        grid=(GRID,),
        in_specs=[_row_spec(x.shape[1]), _full_spec(*w.shape),
                  _full_spec(1, F), _row_spec(1)],
        out_specs=_row_spec(F),
        out_shape=jax.ShapeDtypeStruct((NN, F), jnp.float32),
    )(x, w, b, degs)


def _tc_mid(agg3, degd, w, b, degs):
    return pl.pallas_call(
        _tc_mid_body,
        grid=(GRID,),
        in_specs=[_agg_spec, _row_spec(1), _full_spec(F, F),
                  _full_spec(1, F), _row_spec(1)],
        out_specs=_row_spec(F),
        out_shape=jax.ShapeDtypeStruct((NN, F), jnp.float32),
    )(agg3, degd, w, b, degs)


def _tc_fin(agg3, degd, w2, b2, wd, bd, wc, bc):
    return pl.pallas_call(
        _tc_fin_body,
        grid=(GRID,),
        in_specs=[_agg_spec, _row_spec(1), _full_spec(F, F), _full_spec(1, F),
                  _full_spec(F, F), _full_spec(1, F), _full_spec(F, 45),
                  _full_spec(1, 45)],
        out_specs=_row_spec(45),
        out_shape=jax.ShapeDtypeStruct((NN, 45), jnp.float32),
    )(agg3, degd, w2, b2, wd, bd, wc, bc)


def kernel(x, edge_index, W_in, b_in, W0, b0, W1, b1, W2, b2, Wd, bd, Wc, bc):
    src = edge_index[0].astype(jnp.int32)
    dst = edge_index[1].astype(jnp.int32)
    pad = EP - EE
    srcg = jnp.concatenate([src, jnp.zeros((pad,), jnp.int32)]).reshape(RT, LANE)
    srch = jnp.concatenate([src, jnp.full((pad,), NN, jnp.int32)]).reshape(RT, LANE)
    dsth = jnp.concatenate([dst, jnp.full((pad,), NN, jnp.int32)]).reshape(RT, LANE)
    zh = jnp.zeros((HIST,), jnp.float32)
    za = jnp.zeros((ACK, F), jnp.float32)

    deg2, cg, cl, cnt = _sc_pre(srch, dsth, srcg, zh)
    out_deg = deg2[0, :NN].reshape(NN, 1)
    in_deg = deg2[1, :NN].reshape(NN, 1)

    hs = _tc_in(x, W_in, b_in.reshape(1, F), out_deg)
    for (w, b) in ((W0, b0), (W1, b1)):
        agg3 = _sc_agg(hs, cg, cl, cnt, za)
        hs = _tc_mid(agg3, in_deg, w, b.reshape(1, F), out_deg)
    agg3 = _sc_agg(hs, cg, cl, cnt, za)
    return _tc_fin(agg3, in_deg, W2, b2.reshape(1, F), Wd, bd.reshape(1, F),
                   Wc, bc.reshape(1, 45))
